# Initial kernel scaffold; baseline (speedup 1.0000x reference)
import jax
import jax.numpy as jnp
from jax.experimental import pallas as pl


def kernel(iuv, layer1, layer2, layer3, layer4):
    raise NotImplementedError("write your pallas kernel here")


# speedup vs baseline: 47.2639x; 47.2639x over previous
import functools

import jax
import jax.numpy as jnp
from jax import lax
from jax.experimental import pallas as pl
from jax.experimental.pallas import tpu as pltpu
from jax.experimental.pallas import tpu_sc as plsc

_NC, _NS, _L = 2, 16, 16


@functools.partial(jax.jit, static_argnames=("B", "P", "Ho", "Wo", "N", "levels"))
def _atlas_sc(u, v, t1, t2, t3, t4, *, B, P, Ho, Wo, N, levels):
    NW = _NC * _NS
    G = B * Ho * Wo * P
    GW = G // NW
    PB = 16 * P
    OPB = PB // P
    NBLK = GW // PB
    NVEC = PB // _L

    mesh = plsc.VectorSubcoreMesh(
        core_axis_name="c", subcore_axis_name="s",
        num_cores=_NC, num_subcores=_NS)

    @functools.partial(
        pl.kernel,
        out_type=jax.ShapeDtypeStruct((B * Ho * Wo, N), jnp.float32),
        mesh=mesh,
        scratch_types=[
            pltpu.VMEM((PB,), jnp.float32),
            pltpu.VMEM((PB,), jnp.float32),
            pltpu.VMEM((PB,), jnp.int32),
            pltpu.VMEM((PB,), jnp.int32),
            pltpu.VMEM((PB,), jnp.int32),
            pltpu.VMEM((PB,), jnp.int32),
            pltpu.VMEM((PB,), jnp.float32),
            pltpu.VMEM((PB,), jnp.float32),
            pltpu.VMEM((PB,), jnp.float32),
            pltpu.VMEM((PB,), jnp.float32),
            pltpu.VMEM((PB, N), jnp.float32),
            pltpu.VMEM((PB, N), jnp.float32),
            pltpu.VMEM((PB, N), jnp.float32),
            pltpu.VMEM((PB, N), jnp.float32),
            pltpu.VMEM((OPB, N), jnp.float32),
            pltpu.SemaphoreType.DMA,
        ],
        compiler_params=pltpu.CompilerParams(
            needs_layout_passes=False, use_tc_tiling_on_sc=False),
    )
    def k(u_hbm, v_hbm, x1, x2, x3, x4, out_hbm,
          u_v, v_v, i00, i01, i10, i11, w00, w01, w10, w11,
          r00, r01, r10, r11, ob, sem):
        wid = lax.axis_index("c") * _NS + lax.axis_index("s")
        texs = (x1, x2, x3, x4)
        lane = lax.iota(jnp.int32, _L)

        def block(blk, carry):
            base = pl.multiple_of(wid * GW + blk * PB, PB)
            pltpu.sync_copy(u_hbm.at[pl.ds(base, PB)], u_v)
            pltpu.sync_copy(v_hbm.at[pl.ds(base, PB)], v_v)
            for l, (Hl, Wl) in enumerate(levels):
                tex = texs[l]

                def cw(i16, c2):
                    off = i16 * _L
                    u16 = u_v[pl.ds(off, _L)]
                    v16 = v_v[pl.ds(off, _L)]
                    p16 = (off + lane) % P
                    x = (u16 + 1.0) * (0.5 * (Wl - 1))
                    y = (v16 + 1.0) * (0.5 * (Hl - 1))
                    xi = jnp.clip(x.astype(jnp.int32), 0, Wl - 2)
                    yi = jnp.clip(y.astype(jnp.int32), 0, Hl - 2)
                    fx = x - xi.astype(jnp.float32)
                    fy = y - yi.astype(jnp.float32)
                    rbase = (p16 * Hl + yi) * Wl + xi
                    i00[pl.ds(off, _L)] = rbase
                    i01[pl.ds(off, _L)] = rbase + 1
                    i10[pl.ds(off, _L)] = rbase + Wl
                    i11[pl.ds(off, _L)] = rbase + (Wl + 1)
                    gx = 1.0 - fx
                    gy = 1.0 - fy
                    w00[pl.ds(off, _L)] = gy * gx
                    w01[pl.ds(off, _L)] = gy * fx
                    w10[pl.ds(off, _L)] = fy * gx
                    w11[pl.ds(off, _L)] = fy * fx
                    return c2

                lax.fori_loop(0, NVEC, cw, 0)

                copies = []
                for ir, rr in ((i00, r00), (i01, r01), (i10, r10), (i11, r11)):
                    for c in range(PB // 128):
                        copies.append(pltpu.async_copy(
                            tex.at[ir.at[pl.ds(c * 128, 128)]],
                            rr.at[pl.ds(c * 128, 128), :], sem))
                for cp in copies:
                    cp.wait()

                def op_body(op, c2):
                    def j_body(j, acc):
                        ii = jnp.full((_L,), op * P + j, jnp.int32)
                        wa = plsc.load_gather(w00, [ii])
                        wb = plsc.load_gather(w01, [ii])
                        wc = plsc.load_gather(w10, [ii])
                        wd = plsc.load_gather(w11, [ii])
                        ra = plsc.load_gather(r00, [ii, lane])
                        rb = plsc.load_gather(r01, [ii, lane])
                        rc = plsc.load_gather(r10, [ii, lane])
                        rd = plsc.load_gather(r11, [ii, lane])
                        return acc + wa * ra + wb * rb + wc * rc + wd * rd

                    acc = lax.fori_loop(0, P, j_body,
                                        jnp.zeros((_L,), jnp.float32))
                    if l == 0:
                        ob[op] = acc
                    else:
                        ob[op] = ob[op] + acc
                    return c2

                lax.fori_loop(0, OPB, op_body, 0)
            pltpu.sync_copy(
                ob, out_hbm.at[pl.ds(pl.multiple_of(base // P, OPB), OPB), :])
            return carry

        lax.fori_loop(0, NBLK, block, 0)

    return k(u, v, t1, t2, t3, t4)


def kernel(iuv, layer1, layer2, layer3, layer4):
    B, P, Ho, Wo, _ = iuv.shape
    N = layer1.shape[1]
    layers = (layer1, layer2, layer3, layer4)
    levels = tuple((l.shape[2], l.shape[3]) for l in layers)
    g = jnp.transpose(iuv, (0, 2, 3, 1, 4))
    u = g[..., 0].reshape(-1)
    v = g[..., 1].reshape(-1)
    texs = [jnp.transpose(l, (0, 2, 3, 1)).reshape(-1, N) for l in layers]
    out = _atlas_sc(u, v, *texs, B=B, P=P, Ho=Ho, Wo=Wo, N=N, levels=levels)
    return out.reshape(B, Ho, Wo, N).transpose(0, 3, 1, 2)
